# Initial kernel scaffold; baseline (speedup 1.0000x reference)
import jax
import jax.numpy as jnp
from jax.experimental import pallas as pl


def kernel(node_rep, edge_attr, degree, W1, g1, b1, W2, g2, b2, epsilon, edge_index):
    raise NotImplementedError("write your pallas kernel here")


# speedup vs baseline: 6.5936x; 6.5936x over previous
import functools

import jax
import jax.numpy as jnp
from jax import lax
from jax.experimental import pallas as pl
from jax.experimental.pallas import tpu as pltpu
from jax.experimental.pallas import tpu_sc as plsc

_N = 10000
_E = 320000
_D = 128
_NC = 2
_NS = 16
_NW = _NC * _NS
_CHUNK = 128
_NCHUNKS = _E // _CHUNK
_TRIPS = -(-_NCHUNKS // _NW)
_ROWS_PER_TILE = 624
_TAIL_ROW0 = _NS * _ROWS_PER_TILE
_TAIL_ROWS = _N - _TAIL_ROW0
_BN_EPS = 1e-5


def _sc_segment_sum(x, edge_attr, src, dst, zeros):
    mesh = plsc.VectorSubcoreMesh(core_axis_name="c", subcore_axis_name="s")

    @functools.partial(
        pl.kernel,
        out_type=jax.ShapeDtypeStruct((_NC, _N, _D), jnp.float32),
        mesh=mesh,
        scratch_types=[
            pltpu.VMEM((_CHUNK,), jnp.int32),
            pltpu.VMEM((_CHUNK,), jnp.int32),
            pltpu.VMEM((_CHUNK, _D), jnp.float32),
            pltpu.VMEM((_CHUNK, _D), jnp.float32),
            pltpu.VMEM_SHARED((_N, _D), jnp.float32),
            pltpu.SemaphoreType.DMA,
            pltpu.SemaphoreType.DMA,
        ],
    )
    def k(x_hbm, ea_hbm, src_hbm, dst_hbm, z_hbm, out_hbm,
          src_v, dst_v, xbuf, ebuf, acc, sem0, sem1):
        c = lax.axis_index("c")
        s = lax.axis_index("s")
        wid = c * _NS + s

        row0 = s * _ROWS_PER_TILE
        pltpu.sync_copy(z_hbm.at[pl.ds(row0, _ROWS_PER_TILE)],
                        acc.at[pl.ds(row0, _ROWS_PER_TILE)])

        @pl.when(s == _NS - 1)
        def _():
            pltpu.sync_copy(z_hbm.at[pl.ds(_TAIL_ROW0, _TAIL_ROWS)],
                            acc.at[pl.ds(_TAIL_ROW0, _TAIL_ROWS)])

        plsc.subcore_barrier()

        def body(j, carry):
            cid = wid + j * _NW

            @pl.when(cid < _NCHUNKS)
            def _():
                off = pl.multiple_of(cid * _CHUNK, _CHUNK)
                pltpu.sync_copy(src_hbm.at[pl.ds(off, _CHUNK)], src_v)
                pltpu.sync_copy(dst_hbm.at[pl.ds(off, _CHUNK)], dst_v)
                gat = pltpu.async_copy(x_hbm.at[src_v], xbuf, sem0)
                ecp = pltpu.async_copy(ea_hbm.at[pl.ds(off, _CHUNK)], ebuf, sem1)
                gat.wait()
                ecp.wait()
                pltpu.sync_copy(xbuf, acc.at[dst_v], add=True)
                pltpu.sync_copy(ebuf, acc.at[dst_v], add=True)

            return carry

        lax.fori_loop(0, _TRIPS, body, 0)

        plsc.subcore_barrier()
        pltpu.sync_copy(acc.at[pl.ds(row0, _ROWS_PER_TILE)],
                        out_hbm.at[c].at[pl.ds(row0, _ROWS_PER_TILE)])

        @pl.when(s == _NS - 1)
        def _():
            pltpu.sync_copy(acc.at[pl.ds(_TAIL_ROW0, _TAIL_ROWS)],
                            out_hbm.at[c].at[pl.ds(_TAIL_ROW0, _TAIL_ROWS)])

    return k(x, edge_attr, src, dst, zeros)


def _tc_mlp(parts, x, w1, g1, b1, w2, g2, b2, eps):
    def body(p_ref, x_ref, w1_ref, g1_ref, b1_ref, w2_ref, g2_ref, b2_ref,
             eps_ref, o_ref):
        scale = 1.0 + eps_ref[...]
        h = p_ref[0] + p_ref[1] + scale * x_ref[...]
        z = jnp.dot(h, w1_ref[...], preferred_element_type=jnp.float32)
        mu = jnp.mean(z, axis=0, keepdims=True)
        zc = z - mu
        var = jnp.mean(zc * zc, axis=0, keepdims=True)
        a = jnp.maximum(g1_ref[...] * zc * lax.rsqrt(var + _BN_EPS)
                        + b1_ref[...], 0.0)
        z2 = jnp.dot(a, w2_ref[...], preferred_element_type=jnp.float32)
        mu2 = jnp.mean(z2, axis=0, keepdims=True)
        zc2 = z2 - mu2
        var2 = jnp.mean(zc2 * zc2, axis=0, keepdims=True)
        o_ref[...] = jnp.maximum(g2_ref[...] * zc2 * lax.rsqrt(var2 + _BN_EPS)
                                 + b2_ref[...], 0.0)

    return pl.pallas_call(
        body,
        out_shape=jax.ShapeDtypeStruct((_N, _D), jnp.float32),
    )(parts, x, w1, g1, b1, w2, g2, b2, eps)


def kernel(node_rep, edge_attr, degree, W1, g1, b1, W2, g2, b2, epsilon,
           edge_index):
    del degree
    src = edge_index[0].astype(jnp.int32)
    dst = edge_index[1].astype(jnp.int32)
    zeros = jnp.zeros((_N, _D), jnp.float32)
    parts = _sc_segment_sum(node_rep, edge_attr, src, dst, zeros)
    return _tc_mlp(parts, node_rep,
                   W1, g1.reshape(1, -1), b1.reshape(1, -1),
                   W2, g2.reshape(1, -1), b2.reshape(1, -1),
                   epsilon.reshape(1, 1))
